# baseline (speedup 1.0000x reference)
import jax
import jax.numpy as jnp
from jax import lax
from jax.experimental import pallas as pl
from jax.experimental.pallas import tpu as pltpu
from jax.experimental.pallas import tpu_sc as plsc

N = 10000
E = 320000
H = 128
D_IN = 258
D_PAD = 264
NC = 2
NS = 16
NW = NC * NS
W_WIN = 128
N_WIN = 81
N_ST = N_WIN + 2
EW = N_WIN * W_WIN
E_PAD = NW * EW
DUMMY = N
N_PAD = 10112
R_SUB = N_PAD // NS
R_CH = R_SUB // W_WIN
R_TAIL = R_SUB % W_WIN
BM = 2000


def _proj_body(inp_ref, w_ref, be_ref, ps_ref, pd_ref, q_ref):
    y = jnp.dot(inp_ref[...], w_ref[...],
                preferred_element_type=jnp.float32,
                precision=lax.Precision.HIGHEST)
    ps_ref[...] = y[:, 0:H]
    pd_ref[...] = y[:, H:2 * H] + be_ref[...]
    q_ref[...] = y[:, 2 * H:3 * H]


def _node_body(q_ref, agg_ref, w_ref, bn_ref, o_ref):
    agg = agg_ref[0] + agg_ref[1]
    y = q_ref[...] + jnp.dot(agg, w_ref[...],
                             preferred_element_type=jnp.float32,
                             precision=lax.Precision.HIGHEST) + bn_ref[...]
    o_ref[...] = jnp.maximum(y, 0.0)


def _sc_agg_body(ps_hbm, pd_hbm, src_hbm, dst_hbm, out_hbm,
                 si, di, buf, agg,
                 semA0, semA1, semA2, semB0, semB1, semB2):
    c = lax.axis_index("c")
    s = lax.axis_index("s")
    wid = c * NS + s
    semA = (semA0, semA1, semA2)
    semB = (semB0, semB1, semB2)
    HW = W_WIN // 2

    zv = jnp.zeros((16,), jnp.float32)

    @pl.loop(0, W_WIN)
    def _zero_buf(r):
        for k in range(H // 16):
            buf[0, r, pl.ds(k * 16, 16)] = zv

    base_r = s * R_SUB
    for t in range(R_CH):
        pltpu.sync_copy(buf.at[0], agg.at[pl.ds(base_r + t * W_WIN, W_WIN)])
    pltpu.sync_copy(buf.at[0, pl.ds(0, R_TAIL)],
                    agg.at[pl.ds(base_r + R_CH * W_WIN, R_TAIL)])
    plsc.subcore_barrier()

    def issue_idx(w, k):
        pltpu.async_copy(src_hbm.at[wid, w], si.at[k], semB[k])
        pltpu.async_copy(dst_hbm.at[wid, w], di.at[k], semB[k])

    def wait_idx(k):
        pltpu.make_async_copy(src_hbm.at[wid, 0], si.at[k], semB[k]).wait()
        pltpu.make_async_copy(dst_hbm.at[wid, 0], di.at[k], semB[k]).wait()

    def issue_g(k, h):
        sem = semA[k] if h == 0 else semB[k]
        pltpu.async_copy(pd_hbm.at[di.at[k, pl.ds(h * HW, HW)]],
                         buf.at[k, pl.ds(h * HW, HW)], sem)

    def issue_a(k, h):
        sem = semA[k] if h == 0 else semB[k]
        pltpu.async_copy(ps_hbm.at[si.at[k, pl.ds(h * HW, HW)]],
                         buf.at[k, pl.ds(h * HW, HW)], sem, add=True)

    def wait_half(k, h):
        sem = semA[k] if h == 0 else semB[k]
        pltpu.make_async_copy(ps_hbm.at[pl.ds(0, HW)],
                              buf.at[k, pl.ds(0, HW)], sem).wait()

    def issue_scatter(k):
        pltpu.async_copy(buf.at[k], agg.at[di.at[k]], semA[k], add=True)

    def wait_scatter(k):
        pltpu.make_async_copy(buf.at[k], agg.at[pl.ds(0, W_WIN)],
                              semA[k]).wait()

    def relu(k):
        @pl.loop(0, W_WIN)
        def _relu(r):
            for qq in range(H // 16):
                sl = pl.ds(qq * 16, 16)
                buf[k, r, sl] = jnp.maximum(buf[k, r, sl], 0.0)

    issue_idx(0, 0)
    issue_idx(1, 1)
    wait_idx(0)
    issue_g(0, 0)
    issue_g(0, 1)
    wait_idx(1)
    issue_g(1, 0)
    issue_g(1, 1)
    wait_half(0, 0)
    issue_a(0, 0)
    wait_half(0, 1)
    issue_a(0, 1)
    pltpu.async_copy(agg.at[pl.ds(0, W_WIN)], buf.at[2], semA[2])

    @pl.loop(0, N_WIN // 3)
    def _triple(t):
        w0 = t * 3
        for j in range(3):
            w = w0 + j
            p, q, r = j, (j + 1) % 3, (j + 2) % 3
            wait_half(p, 0)
            wait_half(p, 1)
            wait_scatter(r)
            issue_idx(w + 2, r)
            wait_half(q, 0)
            issue_a(q, 0)
            wait_half(q, 1)
            issue_a(q, 1)
            relu(p)
            issue_scatter(p)
            wait_idx(r)
            issue_g(r, 0)
            issue_g(r, 1)

    wait_half(0, 0)
    wait_half(0, 1)
    wait_half(1, 0)
    wait_half(1, 1)
    wait_scatter(2)
    plsc.subcore_barrier()

    for t in range(R_CH):
        r0 = base_r + t * W_WIN
        pltpu.sync_copy(agg.at[pl.ds(r0, W_WIN)], buf.at[0])
        pltpu.sync_copy(buf.at[0], out_hbm.at[c, pl.ds(r0, W_WIN)])
    r0 = base_r + R_CH * W_WIN
    pltpu.sync_copy(agg.at[pl.ds(r0, R_TAIL)], buf.at[0, pl.ds(0, R_TAIL)])
    pltpu.sync_copy(buf.at[0, pl.ds(0, R_TAIL)],
                    out_hbm.at[c, pl.ds(r0, R_TAIL)])


@jax.jit
def kernel(h, x, pos, edge_index, W_edge, b_edge, W_node, b_node):
    f32 = jnp.float32
    inp = jnp.concatenate(
        [h, x, pos, jnp.zeros((N, D_PAD - D_IN), f32)], axis=-1)
    w_all = jnp.concatenate(
        [W_edge[:D_IN], W_edge[D_IN:], W_node[:D_IN]], axis=1)
    w_all = jnp.concatenate(
        [w_all, jnp.zeros((D_PAD - D_IN, 3 * H), f32)], axis=0)
    be = b_edge.reshape(1, H)
    bn = b_node.reshape(1, H)
    w_nb = W_node[D_IN:]

    ps, pd, q = pl.pallas_call(
        _proj_body,
        grid=(N // BM,),
        in_specs=[
            pl.BlockSpec((BM, D_PAD), lambda i: (i, 0)),
            pl.BlockSpec((D_PAD, 3 * H), lambda i: (0, 0)),
            pl.BlockSpec((1, H), lambda i: (0, 0)),
        ],
        out_specs=[
            pl.BlockSpec((BM, H), lambda i: (i, 0)),
            pl.BlockSpec((BM, H), lambda i: (i, 0)),
            pl.BlockSpec((BM, H), lambda i: (i, 0)),
        ],
        out_shape=[
            jax.ShapeDtypeStruct((N_PAD, H), f32),
            jax.ShapeDtypeStruct((N_PAD, H), f32),
            jax.ShapeDtypeStruct((N, H), f32),
        ],
    )(inp, w_all, be)

    pad = jnp.full((E_PAD - E,), DUMMY, jnp.int32)
    look = jnp.full((NW, 2, W_WIN), DUMMY, jnp.int32)
    src3 = jnp.concatenate(
        [jnp.concatenate([edge_index[0], pad]).reshape(NW, N_WIN, W_WIN),
         look], axis=1)
    dst3 = jnp.concatenate(
        [jnp.concatenate([edge_index[1], pad]).reshape(NW, N_WIN, W_WIN),
         look], axis=1)

    sc_agg = pl.kernel(
        _sc_agg_body,
        out_type=jax.ShapeDtypeStruct((NC, N_PAD, H), f32),
        mesh=plsc.VectorSubcoreMesh(core_axis_name="c", subcore_axis_name="s"),
        scratch_types=[
            pltpu.VMEM((3, W_WIN), jnp.int32),
            pltpu.VMEM((3, W_WIN), jnp.int32),
            pltpu.VMEM((3, W_WIN, H), f32),
            pltpu.VMEM_SHARED((N_PAD, H), f32),
            pltpu.SemaphoreType.DMA,
            pltpu.SemaphoreType.DMA,
            pltpu.SemaphoreType.DMA,
            pltpu.SemaphoreType.DMA,
            pltpu.SemaphoreType.DMA,
            pltpu.SemaphoreType.DMA,
        ],
    )
    aggs = sc_agg(ps, pd, src3, dst3)

    uh = pl.pallas_call(
        _node_body,
        grid=(N // BM,),
        in_specs=[
            pl.BlockSpec((BM, H), lambda i: (i, 0)),
            pl.BlockSpec((NC, BM, H), lambda i: (0, i, 0)),
            pl.BlockSpec((H, H), lambda i: (0, 0)),
            pl.BlockSpec((1, H), lambda i: (0, 0)),
        ],
        out_specs=pl.BlockSpec((BM, H), lambda i: (i, 0)),
        out_shape=jax.ShapeDtypeStruct((N, H), f32),
    )(q, aggs, w_nb, bn)
    return uh


# speedup vs baseline: 4.0215x; 4.0215x over previous
import jax
import jax.numpy as jnp
from jax import lax
from jax.experimental import pallas as pl
from jax.experimental.pallas import tpu as pltpu
from jax.experimental.pallas import tpu_sc as plsc

N = 10000
E = 320000
H = 128
D_IN = 258
D_PAD = 264
NC = 2
NS = 16
NW = NC * NS
W_WIN = 128
N_WIN = 81
N_ST = N_WIN + 2
EW = N_WIN * W_WIN
E_PAD = NW * EW
DUMMY = N
N_PAD = 10112
R_SUB = N_PAD // NS
R_CH = R_SUB // W_WIN
R_TAIL = R_SUB % W_WIN
BM = 2000


def _proj_body(inp_ref, w_ref, be_ref, ps_ref, pd_ref, q_ref):
    y = jnp.dot(inp_ref[...], w_ref[...],
                preferred_element_type=jnp.float32,
                precision=lax.Precision.HIGHEST)
    ps_ref[...] = y[:, 0:H]
    pd_ref[...] = y[:, H:2 * H] + be_ref[...]
    q_ref[...] = y[:, 2 * H:3 * H]


def _node_body(q_ref, agg_ref, w_ref, bn_ref, o_ref):
    agg = agg_ref[0] + agg_ref[1]
    y = q_ref[...] + jnp.dot(agg, w_ref[...],
                             preferred_element_type=jnp.float32,
                             precision=lax.Precision.HIGHEST) + bn_ref[...]
    o_ref[...] = jnp.maximum(y, 0.0)


def _sc_agg_body(ps_hbm, pd_hbm, src_hbm, dst_hbm, out_hbm,
                 si, di, buf, agg,
                 semA0, semA1, semA2, semB0, semB1, semB2):
    c = lax.axis_index("c")
    s = lax.axis_index("s")
    wid = c * NS + s
    semA = (semA0, semA1, semA2)
    semB = (semB0, semB1, semB2)
    HW = W_WIN // 2

    zv = jnp.zeros((16,), jnp.float32)

    @pl.loop(0, W_WIN)
    def _zero_buf(r):
        for k in range(H // 16):
            buf[0, r, pl.ds(k * 16, 16)] = zv

    base_r = s * R_SUB
    for t in range(R_CH):
        pltpu.sync_copy(buf.at[0], agg.at[pl.ds(base_r + t * W_WIN, W_WIN)])
    pltpu.sync_copy(buf.at[0, pl.ds(0, R_TAIL)],
                    agg.at[pl.ds(base_r + R_CH * W_WIN, R_TAIL)])
    plsc.subcore_barrier()

    def issue_idx(w, k):
        pltpu.async_copy(src_hbm.at[wid, w], si.at[k], semB[k])
        pltpu.async_copy(dst_hbm.at[wid, w], di.at[k], semB[k])

    def wait_idx(k):
        pltpu.make_async_copy(src_hbm.at[wid, 0], si.at[k], semB[k]).wait()
        pltpu.make_async_copy(dst_hbm.at[wid, 0], di.at[k], semB[k]).wait()

    def issue_g(k, h):
        sem = semA[k] if h == 0 else semB[k]
        pltpu.async_copy(pd_hbm.at[di.at[k, pl.ds(h * HW, HW)]],
                         buf.at[k, pl.ds(h * HW, HW)], sem)

    def issue_a(k, h):
        sem = semA[k] if h == 0 else semB[k]
        pltpu.async_copy(ps_hbm.at[si.at[k, pl.ds(h * HW, HW)]],
                         buf.at[k, pl.ds(h * HW, HW)], sem, add=True)

    def wait_half(k, h):
        sem = semA[k] if h == 0 else semB[k]
        pltpu.make_async_copy(ps_hbm.at[pl.ds(0, HW)],
                              buf.at[k, pl.ds(0, HW)], sem).wait()

    def issue_scatter(k):
        pltpu.async_copy(buf.at[k], agg.at[di.at[k]], semA[k], add=True)

    def wait_scatter(k):
        pltpu.make_async_copy(buf.at[k], agg.at[pl.ds(0, W_WIN)],
                              semA[k]).wait()

    def relu(k):
        @pl.loop(0, W_WIN)
        def _relu(r):
            for qq in range(H // 16):
                sl = pl.ds(qq * 16, 16)
                buf[k, r, sl] = jnp.maximum(buf[k, r, sl], 0.0)

    issue_idx(0, 0)
    issue_idx(1, 1)
    wait_idx(0)
    issue_g(0, 0)
    issue_g(0, 1)
    wait_idx(1)
    issue_g(1, 0)
    issue_g(1, 1)
    wait_half(0, 0)
    issue_a(0, 0)
    wait_half(0, 1)
    issue_a(0, 1)
    pltpu.async_copy(agg.at[pl.ds(0, W_WIN)], buf.at[2], semA[2])

    @pl.loop(0, N_WIN // 3)
    def _triple(t):
        w0 = t * 3
        for j in range(3):
            w = w0 + j
            p, q, r = j, (j + 1) % 3, (j + 2) % 3
            wait_half(p, 0)
            wait_half(p, 1)
            wait_scatter(r)
            issue_idx(w + 2, r)
            wait_half(q, 0)
            issue_a(q, 0)
            wait_half(q, 1)
            issue_a(q, 1)
            relu(p)
            issue_scatter(p)
            wait_idx(r)
            issue_g(r, 0)
            issue_g(r, 1)

    wait_half(0, 0)
    wait_half(0, 1)
    wait_half(1, 0)
    wait_half(1, 1)
    wait_scatter(2)
    plsc.subcore_barrier()

    for t in range(R_CH):
        r0 = base_r + t * W_WIN
        pltpu.sync_copy(agg.at[pl.ds(r0, W_WIN)], buf.at[0])
        pltpu.sync_copy(buf.at[0], out_hbm.at[c, pl.ds(r0, W_WIN)])
    r0 = base_r + R_CH * W_WIN
    pltpu.sync_copy(agg.at[pl.ds(r0, R_TAIL)], buf.at[0, pl.ds(0, R_TAIL)])
    pltpu.sync_copy(buf.at[0, pl.ds(0, R_TAIL)],
                    out_hbm.at[c, pl.ds(r0, R_TAIL)])


@jax.jit
def kernel(h, x, pos, edge_index, W_edge, b_edge, W_node, b_node):
    f32 = jnp.float32
    inp = jnp.concatenate(
        [h, x, pos, jnp.zeros((N, D_PAD - D_IN), f32)], axis=-1)
    w_all = jnp.concatenate(
        [W_edge[:D_IN], W_edge[D_IN:], W_node[:D_IN]], axis=1)
    w_all = jnp.concatenate(
        [w_all, jnp.zeros((D_PAD - D_IN, 3 * H), f32)], axis=0)
    be = b_edge.reshape(1, H)
    bn = b_node.reshape(1, H)
    w_nb = W_node[D_IN:]

    ps, pd, q = pl.pallas_call(
        _proj_body,
        grid=(N // BM,),
        in_specs=[
            pl.BlockSpec((BM, D_PAD), lambda i: (i, 0)),
            pl.BlockSpec((D_PAD, 3 * H), lambda i: (0, 0)),
            pl.BlockSpec((1, H), lambda i: (0, 0)),
        ],
        out_specs=[
            pl.BlockSpec((BM, H), lambda i: (i, 0)),
            pl.BlockSpec((BM, H), lambda i: (i, 0)),
            pl.BlockSpec((BM, H), lambda i: (i, 0)),
        ],
        out_shape=[
            jax.ShapeDtypeStruct((N_PAD, H), f32),
            jax.ShapeDtypeStruct((N_PAD, H), f32),
            jax.ShapeDtypeStruct((N, H), f32),
        ],
    )(inp, w_all, be)

    n_dummy = N_PAD - N
    pad = DUMMY + jnp.arange(E_PAD - E, dtype=jnp.int32) % n_dummy
    look = (DUMMY + jnp.arange(NW * 2 * W_WIN, dtype=jnp.int32) % n_dummy
            ).reshape(NW, 2, W_WIN)
    src3 = jnp.concatenate(
        [jnp.concatenate([edge_index[0], pad]).reshape(NW, N_WIN, W_WIN),
         look], axis=1)
    dst3 = jnp.concatenate(
        [jnp.concatenate([edge_index[1], pad]).reshape(NW, N_WIN, W_WIN),
         look], axis=1)

    sc_agg = pl.kernel(
        _sc_agg_body,
        out_type=jax.ShapeDtypeStruct((NC, N_PAD, H), f32),
        mesh=plsc.VectorSubcoreMesh(core_axis_name="c", subcore_axis_name="s"),
        scratch_types=[
            pltpu.VMEM((3, W_WIN), jnp.int32),
            pltpu.VMEM((3, W_WIN), jnp.int32),
            pltpu.VMEM((3, W_WIN, H), f32),
            pltpu.VMEM_SHARED((N_PAD, H), f32),
            pltpu.SemaphoreType.DMA,
            pltpu.SemaphoreType.DMA,
            pltpu.SemaphoreType.DMA,
            pltpu.SemaphoreType.DMA,
            pltpu.SemaphoreType.DMA,
            pltpu.SemaphoreType.DMA,
        ],
    )
    aggs = sc_agg(ps, pd, src3, dst3)

    uh = pl.pallas_call(
        _node_body,
        grid=(N // BM,),
        in_specs=[
            pl.BlockSpec((BM, H), lambda i: (i, 0)),
            pl.BlockSpec((NC, BM, H), lambda i: (0, i, 0)),
            pl.BlockSpec((H, H), lambda i: (0, 0)),
            pl.BlockSpec((1, H), lambda i: (0, 0)),
        ],
        out_specs=pl.BlockSpec((BM, H), lambda i: (i, 0)),
        out_shape=jax.ShapeDtypeStruct((N, H), f32),
    )(q, aggs, w_nb, bn)
    return uh


# speedup vs baseline: 4.2893x; 1.0666x over previous
import jax
import jax.numpy as jnp
from jax import lax
from jax.experimental import pallas as pl
from jax.experimental.pallas import tpu as pltpu
from jax.experimental.pallas import tpu_sc as plsc

N = 10000
E = 320000
H = 128
D_IN = 258
D_PAD = 264
NC = 2
NS = 16
NW = NC * NS
W_WIN = 128
N_WIN = 81
N_ST = N_WIN + 2
EW = N_WIN * W_WIN
E_PAD = NW * EW
DUMMY = N
N_PAD = 10112
R_SUB = N_PAD // NS
R_CH = R_SUB // W_WIN
R_TAIL = R_SUB % W_WIN
BM = 2000


def _proj_body(h_ref, x_ref, pos_ref, we_ref, wn_ref, be_ref,
               ps_ref, pd_ref, q_ref):
    inp = jnp.concatenate([h_ref[...], x_ref[...], pos_ref[...]], axis=1)
    w_all = jnp.concatenate(
        [we_ref[0:D_IN], we_ref[D_IN:2 * D_IN], wn_ref[0:D_IN]], axis=1)
    y = jnp.dot(inp, w_all,
                preferred_element_type=jnp.float32,
                precision=lax.Precision.HIGHEST)
    ps_ref[...] = y[:, 0:H]
    pd_ref[...] = y[:, H:2 * H] + be_ref[...]
    q_ref[...] = y[:, 2 * H:3 * H]


def _node_body(q_ref, agg_ref, w_ref, bn_ref, o_ref):
    agg = agg_ref[0] + agg_ref[1]
    y = q_ref[...] + jnp.dot(agg, w_ref[...],
                             preferred_element_type=jnp.float32,
                             precision=lax.Precision.HIGHEST) + bn_ref[...]
    o_ref[...] = jnp.maximum(y, 0.0)


def _sc_agg_body(ps_hbm, pd_hbm, src_hbm, dst_hbm, out_hbm,
                 si, di, buf, agg,
                 semA0, semA1, semA2, semB0, semB1, semB2):
    c = lax.axis_index("c")
    s = lax.axis_index("s")
    wid = c * NS + s
    semA = (semA0, semA1, semA2)
    semB = (semB0, semB1, semB2)
    HW = W_WIN // 2

    zv = jnp.zeros((16,), jnp.float32)

    @pl.loop(0, W_WIN)
    def _zero_buf(r):
        for k in range(H // 16):
            buf[0, r, pl.ds(k * 16, 16)] = zv

    base_r = s * R_SUB
    zcp = []
    for t in range(R_CH):
        zcp.append(pltpu.async_copy(
            buf.at[0], agg.at[pl.ds(base_r + t * W_WIN, W_WIN)], semA[t % 3]))
    zcp.append(pltpu.async_copy(
        buf.at[0, pl.ds(0, R_TAIL)],
        agg.at[pl.ds(base_r + R_CH * W_WIN, R_TAIL)], semB[0]))
    for cp in zcp:
        cp.wait()
    plsc.subcore_barrier()

    wbase = wid * N_WIN

    def issue_idx(w, k):
        pltpu.async_copy(src_hbm.at[wbase + w], si.at[k], semB[k])
        pltpu.async_copy(dst_hbm.at[wbase + w], di.at[k], semB[k])

    def wait_idx(k):
        pltpu.make_async_copy(src_hbm.at[0], si.at[k], semB[k]).wait()
        pltpu.make_async_copy(dst_hbm.at[0], di.at[k], semB[k]).wait()

    def issue_g(k, h):
        sem = semA[k] if h == 0 else semB[k]
        pltpu.async_copy(pd_hbm.at[di.at[k, pl.ds(h * HW, HW)]],
                         buf.at[k, pl.ds(h * HW, HW)], sem)

    def issue_a(k, h):
        sem = semA[k] if h == 0 else semB[k]
        pltpu.async_copy(ps_hbm.at[si.at[k, pl.ds(h * HW, HW)]],
                         buf.at[k, pl.ds(h * HW, HW)], sem, add=True)

    def wait_half(k, h):
        sem = semA[k] if h == 0 else semB[k]
        pltpu.make_async_copy(ps_hbm.at[pl.ds(0, HW)],
                              buf.at[k, pl.ds(0, HW)], sem).wait()

    def issue_scatter(k):
        pltpu.async_copy(buf.at[k], agg.at[di.at[k]], semA[k], add=True)

    def wait_scatter(k):
        pltpu.make_async_copy(buf.at[k], agg.at[pl.ds(0, W_WIN)],
                              semA[k]).wait()

    def relu(k):
        @pl.loop(0, W_WIN, step=2)
        def _relu(r):
            for dr in range(2):
                for qq in range(H // 16):
                    sl = pl.ds(qq * 16, 16)
                    buf[k, r + dr, sl] = jnp.maximum(buf[k, r + dr, sl], 0.0)

    issue_idx(0, 0)
    issue_idx(1, 1)
    wait_idx(0)
    issue_g(0, 0)
    issue_g(0, 1)
    wait_idx(1)
    issue_g(1, 0)
    issue_g(1, 1)
    wait_half(0, 0)
    issue_a(0, 0)
    wait_half(0, 1)
    issue_a(0, 1)
    pltpu.async_copy(agg.at[pl.ds(0, W_WIN)], buf.at[2], semA[2])

    @pl.loop(0, N_WIN // 3)
    def _triple(t):
        w0 = t * 3
        for j in range(3):
            w = w0 + j
            p, q, r = j, (j + 1) % 3, (j + 2) % 3
            wait_half(p, 0)
            wait_half(p, 1)
            wait_scatter(r)
            issue_idx(w + 2, r)
            wait_half(q, 0)
            issue_a(q, 0)
            wait_half(q, 1)
            issue_a(q, 1)
            relu(p)
            issue_scatter(p)
            wait_idx(r)
            issue_g(r, 0)
            issue_g(r, 1)

    wait_half(0, 0)
    wait_half(0, 1)
    wait_half(1, 0)
    wait_half(1, 1)
    wait_scatter(2)
    plsc.subcore_barrier()

    pltpu.sync_copy(agg.at[pl.ds(base_r, R_SUB)],
                    out_hbm.at[c, pl.ds(base_r, R_SUB)])


@jax.jit
def kernel(h, x, pos, edge_index, W_edge, b_edge, W_node, b_node):
    f32 = jnp.float32
    be = b_edge.reshape(1, H)
    bn = b_node.reshape(1, H)
    w_nb = W_node[D_IN:]

    ps, pd, q = pl.pallas_call(
        _proj_body,
        grid=(N // BM,),
        in_specs=[
            pl.BlockSpec((BM, H), lambda i: (i, 0)),
            pl.BlockSpec((BM, H), lambda i: (i, 0)),
            pl.BlockSpec((BM, 2), lambda i: (i, 0)),
            pl.BlockSpec((2 * D_IN, H), lambda i: (0, 0)),
            pl.BlockSpec((D_IN + H, H), lambda i: (0, 0)),
            pl.BlockSpec((1, H), lambda i: (0, 0)),
        ],
        out_specs=[
            pl.BlockSpec((BM, H), lambda i: (i, 0)),
            pl.BlockSpec((BM, H), lambda i: (i, 0)),
            pl.BlockSpec((BM, H), lambda i: (i, 0)),
        ],
        out_shape=[
            jax.ShapeDtypeStruct((N_PAD, H), f32),
            jax.ShapeDtypeStruct((N_PAD, H), f32),
            jax.ShapeDtypeStruct((N, H), f32),
        ],
    )(h, x, pos, W_edge, W_node, be)

    n_dummy = N_PAD - N
    n_pad = E_PAD + 2 * W_WIN - E
    pad = DUMMY + jnp.arange(n_pad, dtype=jnp.int32) % n_dummy
    src3 = jnp.concatenate([edge_index[0], pad]).reshape(-1, W_WIN)
    dst3 = jnp.concatenate([edge_index[1], pad]).reshape(-1, W_WIN)

    sc_agg = pl.kernel(
        _sc_agg_body,
        out_type=jax.ShapeDtypeStruct((NC, N_PAD, H), f32),
        mesh=plsc.VectorSubcoreMesh(core_axis_name="c", subcore_axis_name="s"),
        scratch_types=[
            pltpu.VMEM((3, W_WIN), jnp.int32),
            pltpu.VMEM((3, W_WIN), jnp.int32),
            pltpu.VMEM((3, W_WIN, H), f32),
            pltpu.VMEM_SHARED((N_PAD, H), f32),
            pltpu.SemaphoreType.DMA,
            pltpu.SemaphoreType.DMA,
            pltpu.SemaphoreType.DMA,
            pltpu.SemaphoreType.DMA,
            pltpu.SemaphoreType.DMA,
            pltpu.SemaphoreType.DMA,
        ],
    )
    aggs = sc_agg(ps, pd, src3, dst3)

    uh = pl.pallas_call(
        _node_body,
        grid=(N // BM,),
        in_specs=[
            pl.BlockSpec((BM, H), lambda i: (i, 0)),
            pl.BlockSpec((NC, BM, H), lambda i: (0, i, 0)),
            pl.BlockSpec((H, H), lambda i: (0, 0)),
            pl.BlockSpec((1, H), lambda i: (0, 0)),
        ],
        out_specs=pl.BlockSpec((BM, H), lambda i: (i, 0)),
        out_shape=jax.ShapeDtypeStruct((N, H), f32),
    )(q, aggs, w_nb, bn)
    return uh
